# baseline (speedup 1.0000x reference)
import functools

import jax
import jax.numpy as jnp
from jax import lax
from jax.experimental import pallas as pl
from jax.experimental.pallas import tpu as pltpu
from jax.experimental.pallas import tpu_sc as plsc

RATIO = 0.1
ROWS = 128
FEAT = 32768
NWORK = 32
ROWS_PER_W = ROWS // NWORK
CHUNKS = FEAT // 16
INT_MIN = -2147483648


def _sc_kwta(x_hbm, out_hbm, buf_v, cidx_v, hist_v, tot_v, sem, *, k):
    wid = lax.axis_index("s") * 2 + lax.axis_index("c")
    lane = lax.iota(jnp.int32, 16)
    lane_base = lane * jnp.int32(256)
    lane_cap = lane * jnp.int32(CHUNKS)
    ones16 = jnp.ones((16,), jnp.int32)
    zeros16 = jnp.zeros((16,), jnp.int32)

    def fold_and_scan(krem):
        @plsc.parallel_loop(0, 16, unroll=2)
        def fold_body(i):
            acc = hist_v[pl.ds(i * 16, 16)]
            hist_v[pl.ds(i * 16, 16)] = jnp.zeros((16,), jnp.int32)
            for l in range(1, 16):
                acc = acc + hist_v[pl.ds(l * 256 + i * 16, 16)]
                hist_v[pl.ds(l * 256 + i * 16, 16)] = jnp.zeros((16,),
                                                                jnp.int32)
            tot_v[pl.ds(i * 16, 16)] = acc

        def scan_body(i, carry):
            cum, chosen, kr = carry
            g = jnp.int32(15) - i
            v = tot_v[pl.ds(g * 16, 16)]
            rv = lax.rev(v, (0,))
            cs = plsc.cumsum(rv) + cum
            hit = cs >= kr
            npop = plsc.all_reduce_population_count(hit)
            p = plsc.all_reduce_ffs(hit)
            sel = lane == p
            cum_at = jnp.broadcast_to(
                jnp.sum(jnp.where(sel, cs, jnp.int32(0))), (16,))
            tot_at = jnp.broadcast_to(
                jnp.sum(jnp.where(sel, rv, jnp.int32(0))), (16,))
            found_now = jnp.logical_and(chosen < 0, npop > 0)
            chosen_local = g * 16 + (jnp.int32(15) - p)
            chosen = jnp.where(found_now, chosen_local, chosen)
            kr = jnp.where(found_now, kr - (cum_at - tot_at), kr)
            cum = jnp.broadcast_to(jnp.max(cs), (16,))
            return cum, chosen, kr

        _, chosen, krem = lax.fori_loop(
            0, 16, scan_body,
            (jnp.zeros((16,), jnp.int32),
             jnp.full((16,), -1, jnp.int32), krem))
        return chosen, krem

    @plsc.parallel_loop(0, 256, unroll=8)
    def zero_body(i):
        hist_v[pl.ds(i * 16, 16)] = jnp.zeros((16,), jnp.int32)

    for rr in range(ROWS_PER_W):
        row = wid * ROWS_PER_W + rr
        pltpu.sync_copy(x_hbm.at[row], buf_v)

        @plsc.parallel_loop(0, CHUNKS, unroll=8)
        def histA_body(i):
            xi = buf_v[pl.ds(i * 16, 16)]
            m = xi ^ ((xi >> 31) & jnp.int32(0x7FFFFFFF))
            bucket = ((m >> 24) & jnp.int32(0xFF)) ^ jnp.int32(0x80)
            plsc.addupdate_scatter(hist_v, [lane_base + bucket], ones16)

        krem = jnp.full((16,), k, jnp.int32)
        chosen, krem = fold_and_scan(krem)
        prefix_val = (chosen ^ jnp.int32(0x80)) << 24
        prefix_mask = jnp.full((16,), INT_MIN >> 7, jnp.int32)
        tb_c = prefix_val >> 24

        @plsc.parallel_loop(0, CHUNKS, unroll=4,
                            carry=jnp.zeros((16,), jnp.int32))
        def passB(i, cnt):
            xi = buf_v[pl.ds(i * 16, 16)]
            m = xi ^ ((xi >> 31) & jnp.int32(0x7FFFFFFF))
            tb = m >> 24
            eq = tb == tb_c
            buf_v[pl.ds(i * 16, 16)] = jnp.where(tb >= tb_c, xi,
                                                 jnp.int32(0))
            plsc.store_scatter(cidx_v, [lane_cap + cnt],
                               i * jnp.int32(16) + lane, mask=eq)
            return cnt + jnp.where(eq, jnp.int32(1), jnp.int32(0))

        cnt = passB
        maxcnt = jnp.max(cnt)

        for t in range(1, 4):
            shift = 24 - 8 * t
            pv = prefix_val
            pm = prefix_mask

            def hist_small(i, _, shift=shift, pv=pv, pm=pm, cnt=cnt):
                valid = cnt > i
                pos = plsc.load_gather(cidx_v, [lane_cap + i], mask=valid)
                pos = jnp.where(valid, pos, jnp.int32(0))
                xi = plsc.load_gather(buf_v, [pos], mask=valid)
                m = xi ^ ((xi >> 31) & jnp.int32(0x7FFFFFFF))
                match = jnp.logical_and((m & pm) == pv, valid)
                bucket = (m >> shift) & jnp.int32(0xFF)
                plsc.addupdate_scatter(hist_v, [lane_base + bucket],
                                       ones16, mask=match)
                return _

            lax.fori_loop(0, maxcnt, hist_small, jnp.int32(0))
            chosen, krem = fold_and_scan(krem)
            prefix_val = prefix_val | (chosen << shift)
            mask_c = (0xFF << shift) & 0xFFFFFFFF
            if mask_c >= 2**31:
                mask_c -= 2**32
            prefix_mask = prefix_mask | jnp.full((16,), mask_c, jnp.int32)

        thr_m = prefix_val

        def fixup(i, _, cnt=cnt, thr_m=thr_m):
            valid = cnt > i
            pos = plsc.load_gather(cidx_v, [lane_cap + i], mask=valid)
            pos = jnp.where(valid, pos, jnp.int32(0))
            xi = plsc.load_gather(buf_v, [pos], mask=valid)
            m = xi ^ ((xi >> 31) & jnp.int32(0x7FFFFFFF))
            rej = jnp.logical_and(m < thr_m, valid)
            plsc.store_scatter(buf_v, [pos], zeros16, mask=rej)
            return _

        lax.fori_loop(0, maxcnt, fixup, jnp.int32(0))
        pltpu.sync_copy(buf_v, out_hbm.at[row])


def kernel(inputs):
    rows, features = inputs.shape
    k = max(int(round(RATIO * features)), 1)
    xi = lax.bitcast_convert_type(inputs, jnp.int32)
    mesh = plsc.VectorSubcoreMesh(core_axis_name="c", subcore_axis_name="s")
    out_i = pl.kernel(
        functools.partial(_sc_kwta, k=k),
        mesh=mesh,
        compiler_params=pltpu.CompilerParams(needs_layout_passes=False),
        out_type=jax.ShapeDtypeStruct((rows, features), jnp.int32),
        scratch_types=[
            pltpu.VMEM((FEAT,), jnp.int32),
            pltpu.VMEM((FEAT,), jnp.int32),
            pltpu.VMEM((4096,), jnp.int32),
            pltpu.VMEM((256,), jnp.int32),
            pltpu.SemaphoreType.DMA,
        ],
    )(xi)
    return lax.bitcast_convert_type(out_i, jnp.float32)


# speedup vs baseline: 2.1561x; 2.1561x over previous
import functools

import jax
import jax.numpy as jnp
from jax import lax
from jax.experimental import pallas as pl
from jax.experimental.pallas import tpu as pltpu
from jax.experimental.pallas import tpu_sc as plsc

RATIO = 0.1
ROWS = 128
FEAT = 32768
NWORK = 32
ROWS_PER_W = ROWS // NWORK
CHUNKS = FEAT // 16
INT_MIN = -2147483648


def _sc_kwta(x_hbm, out_hbm, buf_v, cidx_v, cidx2_v, hist_v, tot_v, sem, *,
             k):
    wid = lax.axis_index("s") * 2 + lax.axis_index("c")
    lane = lax.iota(jnp.int32, 16)
    lane_base = lane * jnp.int32(256)
    lane_cap = lane * jnp.int32(CHUNKS)
    ones16 = jnp.ones((16,), jnp.int32)
    zeros16 = jnp.zeros((16,), jnp.int32)

    def fold_and_scan(krem):
        @plsc.parallel_loop(0, 16, unroll=2)
        def fold_body(i):
            acc = hist_v[pl.ds(i * 16, 16)]
            hist_v[pl.ds(i * 16, 16)] = jnp.zeros((16,), jnp.int32)
            for l in range(1, 16):
                acc = acc + hist_v[pl.ds(l * 256 + i * 16, 16)]
                hist_v[pl.ds(l * 256 + i * 16, 16)] = jnp.zeros((16,),
                                                                jnp.int32)
            tot_v[pl.ds(i * 16, 16)] = acc

        def scan_body(i, carry):
            cum, chosen, kr = carry
            g = jnp.int32(15) - i
            v = tot_v[pl.ds(g * 16, 16)]
            rv = lax.rev(v, (0,))
            cs = plsc.cumsum(rv) + cum
            hit = cs >= kr
            npop = plsc.all_reduce_population_count(hit)
            p = plsc.all_reduce_ffs(hit)
            sel = lane == p
            cum_at = jnp.broadcast_to(
                jnp.sum(jnp.where(sel, cs, jnp.int32(0))), (16,))
            tot_at = jnp.broadcast_to(
                jnp.sum(jnp.where(sel, rv, jnp.int32(0))), (16,))
            found_now = jnp.logical_and(chosen < 0, npop > 0)
            chosen_local = g * 16 + (jnp.int32(15) - p)
            chosen = jnp.where(found_now, chosen_local, chosen)
            kr = jnp.where(found_now, kr - (cum_at - tot_at), kr)
            cum = jnp.broadcast_to(jnp.max(cs), (16,))
            return cum, chosen, kr

        _, chosen, krem = lax.fori_loop(
            0, 16, scan_body,
            (jnp.zeros((16,), jnp.int32),
             jnp.full((16,), -1, jnp.int32), krem))
        return chosen, krem

    @plsc.parallel_loop(0, 256, unroll=8)
    def zero_body(i):
        hist_v[pl.ds(i * 16, 16)] = jnp.zeros((16,), jnp.int32)

    for rr in range(ROWS_PER_W):
        row = wid * ROWS_PER_W + rr
        pltpu.sync_copy(x_hbm.at[row], buf_v)

        @plsc.parallel_loop(0, CHUNKS, unroll=8)
        def histA_body(i):
            xi = buf_v[pl.ds(i * 16, 16)]
            m = xi ^ ((xi >> 31) & jnp.int32(0x7FFFFFFF))
            bucket = ((m >> 24) & jnp.int32(0xFF)) ^ jnp.int32(0x80)
            plsc.addupdate_scatter(hist_v, [lane_base + bucket], ones16)

        krem = jnp.full((16,), k, jnp.int32)
        chosen, krem = fold_and_scan(krem)
        prefix_val = (chosen ^ jnp.int32(0x80)) << 24
        prefix_mask = jnp.full((16,), INT_MIN >> 7, jnp.int32)
        tb_c = prefix_val >> 24

        @plsc.parallel_loop(0, CHUNKS, unroll=4,
                            carry=jnp.zeros((16,), jnp.int32))
        def passB(i, cnt):
            xi = buf_v[pl.ds(i * 16, 16)]
            m = xi ^ ((xi >> 31) & jnp.int32(0x7FFFFFFF))
            tb = m >> 24
            eq = tb == tb_c
            buf_v[pl.ds(i * 16, 16)] = jnp.where(tb >= tb_c, xi,
                                                 jnp.int32(0))
            plsc.store_scatter(cidx_v, [lane_cap + cnt],
                               i * jnp.int32(16) + lane, mask=eq)
            bucket2 = (m >> 16) & jnp.int32(0xFF)
            plsc.addupdate_scatter(hist_v, [lane_base + bucket2],
                                   ones16, mask=eq)
            return cnt + jnp.where(eq, jnp.int32(1), jnp.int32(0))

        cnt = passB
        maxcnt = jnp.max(cnt)
        chosen2, krem = fold_and_scan(krem)
        b2_c = jnp.broadcast_to(chosen2, (16,))

        def recomp(j, cnt2):
            for u in range(4):
                i = j * 4 + jnp.int32(u)
                valid = cnt > i
                pos = plsc.load_gather(cidx_v, [lane_cap + i], mask=valid)
                pos = jnp.where(valid, pos, jnp.int32(0))
                xi = plsc.load_gather(buf_v, [pos], mask=valid)
                m = xi ^ ((xi >> 31) & jnp.int32(0x7FFFFFFF))
                b2 = (m >> 16) & jnp.int32(0xFF)
                eq = jnp.logical_and(b2 == b2_c, valid)
                lt = jnp.logical_and(b2 < b2_c, valid)
                plsc.store_scatter(buf_v, [pos], zeros16, mask=lt)
                plsc.store_scatter(cidx2_v, [lane_cap + cnt2], pos,
                                   mask=eq)
                bucket3 = (m >> 8) & jnp.int32(0xFF)
                plsc.addupdate_scatter(hist_v, [lane_base + bucket3],
                                       ones16, mask=eq)
                cnt2 = cnt2 + jnp.where(eq, jnp.int32(1), jnp.int32(0))
            return cnt2

        cnt2 = lax.fori_loop(0, (maxcnt + 3) // 4, recomp,
                             jnp.zeros((16,), jnp.int32))
        maxcnt2 = jnp.max(cnt2)
        chosen3, krem = fold_and_scan(krem)
        prefix_val = (prefix_val | (chosen2 << 16)) | (chosen3 << 8)
        pv3 = prefix_val & jnp.int32(-256)

        def hist4(i, _, cnt2=cnt2, pv3=pv3):
            valid = cnt2 > i
            pos = plsc.load_gather(cidx2_v, [lane_cap + i], mask=valid)
            pos = jnp.where(valid, pos, jnp.int32(0))
            xi = plsc.load_gather(buf_v, [pos], mask=valid)
            m = xi ^ ((xi >> 31) & jnp.int32(0x7FFFFFFF))
            match = jnp.logical_and((m & jnp.int32(-256)) == pv3, valid)
            bucket = m & jnp.int32(0xFF)
            plsc.addupdate_scatter(hist_v, [lane_base + bucket],
                                   ones16, mask=match)
            return _

        lax.fori_loop(0, maxcnt2, hist4, jnp.int32(0))
        chosen4, krem = fold_and_scan(krem)
        thr_m = prefix_val | chosen4

        def fixup(i, _, cnt2=cnt2, thr_m=thr_m):
            valid = cnt2 > i
            pos = plsc.load_gather(cidx2_v, [lane_cap + i], mask=valid)
            pos = jnp.where(valid, pos, jnp.int32(0))
            xi = plsc.load_gather(buf_v, [pos], mask=valid)
            m = xi ^ ((xi >> 31) & jnp.int32(0x7FFFFFFF))
            rej = jnp.logical_and(m < thr_m, valid)
            plsc.store_scatter(buf_v, [pos], zeros16, mask=rej)
            return _

        lax.fori_loop(0, maxcnt2, fixup, jnp.int32(0))
        pltpu.sync_copy(buf_v, out_hbm.at[row])


def kernel(inputs):
    rows, features = inputs.shape
    k = max(int(round(RATIO * features)), 1)
    xi = lax.bitcast_convert_type(inputs, jnp.int32)
    mesh = plsc.VectorSubcoreMesh(core_axis_name="c", subcore_axis_name="s")
    out_i = pl.kernel(
        functools.partial(_sc_kwta, k=k),
        mesh=mesh,
        compiler_params=pltpu.CompilerParams(needs_layout_passes=False),
        out_type=jax.ShapeDtypeStruct((rows, features), jnp.int32),
        scratch_types=[
            pltpu.VMEM((FEAT,), jnp.int32),
            pltpu.VMEM((FEAT,), jnp.int32),
            pltpu.VMEM((FEAT,), jnp.int32),
            pltpu.VMEM((4096,), jnp.int32),
            pltpu.VMEM((256,), jnp.int32),
            pltpu.SemaphoreType.DMA,
        ],
    )(xi)
    return lax.bitcast_convert_type(out_i, jnp.float32)


# speedup vs baseline: 2.1618x; 1.0026x over previous
import functools

import jax
import jax.numpy as jnp
from jax import lax
from jax.experimental import pallas as pl
from jax.experimental.pallas import tpu as pltpu
from jax.experimental.pallas import tpu_sc as plsc

RATIO = 0.1
ROWS = 128
FEAT = 32768
NWORK = 32
ROWS_PER_W = ROWS // NWORK
CHUNKS = FEAT // 16
INT_MIN = -2147483648


def _sc_kwta(x_hbm, out_hbm, buf_v, cidx_v, cidx2_v, hist_v, tot_v, sem, *,
             k):
    wid = lax.axis_index("s") * 2 + lax.axis_index("c")
    lane = lax.iota(jnp.int32, 16)
    lane_base = lane * jnp.int32(256)
    lane_cap = lane * jnp.int32(CHUNKS)
    ones16 = jnp.ones((16,), jnp.int32)
    zeros16 = jnp.zeros((16,), jnp.int32)

    def fold_and_scan(krem):
        @plsc.parallel_loop(0, 16, unroll=2)
        def fold_body(i):
            acc = hist_v[pl.ds(i * 16, 16)]
            hist_v[pl.ds(i * 16, 16)] = jnp.zeros((16,), jnp.int32)
            for l in range(1, 16):
                acc = acc + hist_v[pl.ds(l * 256 + i * 16, 16)]
                hist_v[pl.ds(l * 256 + i * 16, 16)] = jnp.zeros((16,),
                                                                jnp.int32)
            tot_v[pl.ds(i * 16, 16)] = acc

        def scan_body(i, carry):
            cum, chosen, kr = carry
            g = jnp.int32(15) - i
            v = tot_v[pl.ds(g * 16, 16)]
            rv = lax.rev(v, (0,))
            cs = plsc.cumsum(rv) + cum
            hit = cs >= kr
            npop = plsc.all_reduce_population_count(hit)
            p = plsc.all_reduce_ffs(hit)
            sel = lane == p
            cum_at = jnp.broadcast_to(
                jnp.sum(jnp.where(sel, cs, jnp.int32(0))), (16,))
            tot_at = jnp.broadcast_to(
                jnp.sum(jnp.where(sel, rv, jnp.int32(0))), (16,))
            found_now = jnp.logical_and(chosen < 0, npop > 0)
            chosen_local = g * 16 + (jnp.int32(15) - p)
            chosen = jnp.where(found_now, chosen_local, chosen)
            kr = jnp.where(found_now, kr - (cum_at - tot_at), kr)
            cum = jnp.broadcast_to(jnp.max(cs), (16,))
            return cum, chosen, kr

        _, chosen, krem = lax.fori_loop(
            0, 16, scan_body,
            (jnp.zeros((16,), jnp.int32),
             jnp.full((16,), -1, jnp.int32), krem))
        return chosen, krem

    @plsc.parallel_loop(0, 256, unroll=8)
    def zero_body(i):
        hist_v[pl.ds(i * 16, 16)] = jnp.zeros((16,), jnp.int32)

    for rr in range(ROWS_PER_W):
        row = wid * ROWS_PER_W + rr
        pltpu.sync_copy(x_hbm.at[row], buf_v)

        @plsc.parallel_loop(0, CHUNKS, unroll=8)
        def histA_body(i):
            xi = buf_v[pl.ds(i * 16, 16)]
            m = xi ^ ((xi >> 31) & jnp.int32(0x7FFFFFFF))
            bucket = ((m >> 24) & jnp.int32(0xFF)) ^ jnp.int32(0x80)
            plsc.addupdate_scatter(hist_v, [lane_base + bucket], ones16)

        krem = jnp.full((16,), k, jnp.int32)
        chosen, krem = fold_and_scan(krem)
        prefix_val = (chosen ^ jnp.int32(0x80)) << 24
        prefix_mask = jnp.full((16,), INT_MIN >> 7, jnp.int32)
        tb_c = prefix_val >> 24

        @plsc.parallel_loop(0, CHUNKS, unroll=8,
                            carry=jnp.zeros((16,), jnp.int32))
        def passB(i, cnt):
            xi = buf_v[pl.ds(i * 16, 16)]
            m = xi ^ ((xi >> 31) & jnp.int32(0x7FFFFFFF))
            tb = m >> 24
            eq = tb == tb_c
            buf_v[pl.ds(i * 16, 16)] = jnp.where(tb >= tb_c, xi,
                                                 jnp.int32(0))
            plsc.store_scatter(cidx_v, [lane_cap + cnt],
                               i * jnp.int32(16) + lane, mask=eq)
            bucket2 = (m >> 16) & jnp.int32(0xFF)
            plsc.addupdate_scatter(hist_v, [lane_base + bucket2],
                                   ones16, mask=eq)
            return cnt + jnp.where(eq, jnp.int32(1), jnp.int32(0))

        cnt = passB
        maxcnt = jnp.max(cnt)
        chosen2, krem = fold_and_scan(krem)
        b2_c = jnp.broadcast_to(chosen2, (16,))

        def recomp(j, cnt2):
            for u in range(4):
                i = j * 4 + jnp.int32(u)
                valid = cnt > i
                pos = plsc.load_gather(cidx_v, [lane_cap + i], mask=valid)
                pos = jnp.where(valid, pos, jnp.int32(0))
                xi = plsc.load_gather(buf_v, [pos], mask=valid)
                m = xi ^ ((xi >> 31) & jnp.int32(0x7FFFFFFF))
                b2 = (m >> 16) & jnp.int32(0xFF)
                eq = jnp.logical_and(b2 == b2_c, valid)
                lt = jnp.logical_and(b2 < b2_c, valid)
                plsc.store_scatter(buf_v, [pos], zeros16, mask=lt)
                plsc.store_scatter(cidx2_v, [lane_cap + cnt2], pos,
                                   mask=eq)
                bucket3 = (m >> 8) & jnp.int32(0xFF)
                plsc.addupdate_scatter(hist_v, [lane_base + bucket3],
                                       ones16, mask=eq)
                cnt2 = cnt2 + jnp.where(eq, jnp.int32(1), jnp.int32(0))
            return cnt2

        cnt2 = lax.fori_loop(0, (maxcnt + 3) // 4, recomp,
                             jnp.zeros((16,), jnp.int32))
        maxcnt2 = jnp.max(cnt2)
        chosen3, krem = fold_and_scan(krem)
        prefix_val = (prefix_val | (chosen2 << 16)) | (chosen3 << 8)
        pv3 = prefix_val & jnp.int32(-256)

        def hist4(i, _, cnt2=cnt2, pv3=pv3):
            valid = cnt2 > i
            pos = plsc.load_gather(cidx2_v, [lane_cap + i], mask=valid)
            pos = jnp.where(valid, pos, jnp.int32(0))
            xi = plsc.load_gather(buf_v, [pos], mask=valid)
            m = xi ^ ((xi >> 31) & jnp.int32(0x7FFFFFFF))
            match = jnp.logical_and((m & jnp.int32(-256)) == pv3, valid)
            bucket = m & jnp.int32(0xFF)
            plsc.addupdate_scatter(hist_v, [lane_base + bucket],
                                   ones16, mask=match)
            return _

        lax.fori_loop(0, maxcnt2, hist4, jnp.int32(0))
        chosen4, krem = fold_and_scan(krem)
        thr_m = prefix_val | chosen4

        def fixup(i, _, cnt2=cnt2, thr_m=thr_m):
            valid = cnt2 > i
            pos = plsc.load_gather(cidx2_v, [lane_cap + i], mask=valid)
            pos = jnp.where(valid, pos, jnp.int32(0))
            xi = plsc.load_gather(buf_v, [pos], mask=valid)
            m = xi ^ ((xi >> 31) & jnp.int32(0x7FFFFFFF))
            rej = jnp.logical_and(m < thr_m, valid)
            plsc.store_scatter(buf_v, [pos], zeros16, mask=rej)
            return _

        lax.fori_loop(0, maxcnt2, fixup, jnp.int32(0))
        pltpu.sync_copy(buf_v, out_hbm.at[row])


def kernel(inputs):
    rows, features = inputs.shape
    k = max(int(round(RATIO * features)), 1)
    xi = lax.bitcast_convert_type(inputs, jnp.int32)
    mesh = plsc.VectorSubcoreMesh(core_axis_name="c", subcore_axis_name="s")
    out_i = pl.kernel(
        functools.partial(_sc_kwta, k=k),
        mesh=mesh,
        compiler_params=pltpu.CompilerParams(needs_layout_passes=False),
        out_type=jax.ShapeDtypeStruct((rows, features), jnp.int32),
        scratch_types=[
            pltpu.VMEM((FEAT,), jnp.int32),
            pltpu.VMEM((FEAT,), jnp.int32),
            pltpu.VMEM((FEAT,), jnp.int32),
            pltpu.VMEM((4096,), jnp.int32),
            pltpu.VMEM((256,), jnp.int32),
            pltpu.SemaphoreType.DMA,
        ],
    )(xi)
    return lax.bitcast_convert_type(out_i, jnp.float32)


# speedup vs baseline: 2.1742x; 1.0057x over previous
import functools

import jax
import jax.numpy as jnp
from jax import lax
from jax.experimental import pallas as pl
from jax.experimental.pallas import tpu as pltpu
from jax.experimental.pallas import tpu_sc as plsc

RATIO = 0.1
ROWS = 128
FEAT = 32768
NWORK = 32
ROWS_PER_W = ROWS // NWORK
CHUNKS = FEAT // 16
INT_MIN = -2147483648


def _sc_kwta(x_hbm, out_hbm, buf_v, cidx_v, cidx2_v, hist_v, tot_v, sem, *,
             k):
    wid = lax.axis_index("s") * 2 + lax.axis_index("c")
    lane = lax.iota(jnp.int32, 16)
    lane_base = lane * jnp.int32(256)
    lane_cap = lane * jnp.int32(CHUNKS)
    ones16 = jnp.ones((16,), jnp.int32)
    zeros16 = jnp.zeros((16,), jnp.int32)

    def fold_and_scan(krem):
        @plsc.parallel_loop(0, 16, unroll=2)
        def fold_body(i):
            acc = hist_v[pl.ds(i * 16, 16)]
            hist_v[pl.ds(i * 16, 16)] = jnp.zeros((16,), jnp.int32)
            for l in range(1, 16):
                acc = acc + hist_v[pl.ds(l * 256 + i * 16, 16)]
                hist_v[pl.ds(l * 256 + i * 16, 16)] = jnp.zeros((16,),
                                                                jnp.int32)
            tot_v[pl.ds(i * 16, 16)] = acc

        def scan_body(i, carry):
            cum, chosen, kr = carry
            g = jnp.int32(15) - i
            v = tot_v[pl.ds(g * 16, 16)]
            rv = lax.rev(v, (0,))
            cs = plsc.cumsum(rv) + cum
            hit = cs >= kr
            npop = plsc.all_reduce_population_count(hit)
            p = plsc.all_reduce_ffs(hit)
            sel = lane == p
            cum_at = jnp.broadcast_to(
                jnp.sum(jnp.where(sel, cs, jnp.int32(0))), (16,))
            tot_at = jnp.broadcast_to(
                jnp.sum(jnp.where(sel, rv, jnp.int32(0))), (16,))
            found_now = jnp.logical_and(chosen < 0, npop > 0)
            chosen_local = g * 16 + (jnp.int32(15) - p)
            chosen = jnp.where(found_now, chosen_local, chosen)
            kr = jnp.where(found_now, kr - (cum_at - tot_at), kr)
            cum = jnp.broadcast_to(jnp.max(cs), (16,))
            return cum, chosen, kr

        _, chosen, krem = lax.fori_loop(
            0, 16, scan_body,
            (jnp.zeros((16,), jnp.int32),
             jnp.full((16,), -1, jnp.int32), krem))
        return chosen, krem

    @plsc.parallel_loop(0, 256, unroll=8)
    def zero_body(i):
        hist_v[pl.ds(i * 16, 16)] = jnp.zeros((16,), jnp.int32)

    for rr in range(ROWS_PER_W):
        row = wid * ROWS_PER_W + rr
        pltpu.sync_copy(x_hbm.at[row], buf_v)

        @plsc.parallel_loop(0, CHUNKS, unroll=8)
        def histA_body(i):
            xi = buf_v[pl.ds(i * 16, 16)]
            m = xi ^ ((xi >> 31) & jnp.int32(0x7FFFFFFF))
            bucket = ((m >> 24) & jnp.int32(0xFF)) ^ jnp.int32(0x80)
            plsc.addupdate_scatter(hist_v, [lane_base + bucket], ones16)

        krem = jnp.full((16,), k, jnp.int32)
        chosen, krem = fold_and_scan(krem)
        prefix_val = (chosen ^ jnp.int32(0x80)) << 24
        prefix_mask = jnp.full((16,), INT_MIN >> 7, jnp.int32)
        tb_c = prefix_val >> 24

        @plsc.parallel_loop(0, CHUNKS, unroll=8,
                            carry=jnp.zeros((16,), jnp.int32))
        def passB(i, cnt):
            xi = buf_v[pl.ds(i * 16, 16)]
            m = xi ^ ((xi >> 31) & jnp.int32(0x7FFFFFFF))
            tb = m >> 24
            eq = tb == tb_c
            buf_v[pl.ds(i * 16, 16)] = jnp.where(tb >= tb_c, xi,
                                                 jnp.int32(0))
            plsc.store_scatter(cidx_v, [lane_cap + cnt],
                               i * jnp.int32(16) + lane, mask=eq)
            bucket2 = (m >> 16) & jnp.int32(0xFF)
            plsc.addupdate_scatter(hist_v, [lane_base + bucket2],
                                   ones16, mask=eq)
            return cnt + jnp.where(eq, jnp.int32(1), jnp.int32(0))

        cnt = passB
        maxcnt = jnp.max(cnt)
        chosen2, krem = fold_and_scan(krem)
        b2_c = jnp.broadcast_to(chosen2, (16,))

        def recomp(j, cnt2):
            for u in range(8):
                i = j * 8 + jnp.int32(u)
                valid = cnt > i
                pos = plsc.load_gather(cidx_v, [lane_cap + i], mask=valid)
                pos = jnp.where(valid, pos, jnp.int32(0))
                xi = plsc.load_gather(buf_v, [pos], mask=valid)
                m = xi ^ ((xi >> 31) & jnp.int32(0x7FFFFFFF))
                b2 = (m >> 16) & jnp.int32(0xFF)
                eq = jnp.logical_and(b2 == b2_c, valid)
                lt = jnp.logical_and(b2 < b2_c, valid)
                plsc.store_scatter(buf_v, [pos], zeros16, mask=lt)
                plsc.store_scatter(cidx2_v, [lane_cap + cnt2], pos,
                                   mask=eq)
                bucket3 = (m >> 8) & jnp.int32(0xFF)
                plsc.addupdate_scatter(hist_v, [lane_base + bucket3],
                                       ones16, mask=eq)
                cnt2 = cnt2 + jnp.where(eq, jnp.int32(1), jnp.int32(0))
            return cnt2

        cnt2 = lax.fori_loop(0, (maxcnt + 7) // 8, recomp,
                             jnp.zeros((16,), jnp.int32))
        maxcnt2 = jnp.max(cnt2)
        chosen3, krem = fold_and_scan(krem)
        prefix_val = (prefix_val | (chosen2 << 16)) | (chosen3 << 8)
        pv3 = prefix_val & jnp.int32(-256)

        def hist4(i, _, cnt2=cnt2, pv3=pv3):
            valid = cnt2 > i
            pos = plsc.load_gather(cidx2_v, [lane_cap + i], mask=valid)
            pos = jnp.where(valid, pos, jnp.int32(0))
            xi = plsc.load_gather(buf_v, [pos], mask=valid)
            m = xi ^ ((xi >> 31) & jnp.int32(0x7FFFFFFF))
            match = jnp.logical_and((m & jnp.int32(-256)) == pv3, valid)
            bucket = m & jnp.int32(0xFF)
            plsc.addupdate_scatter(hist_v, [lane_base + bucket],
                                   ones16, mask=match)
            return _

        lax.fori_loop(0, maxcnt2, hist4, jnp.int32(0))
        chosen4, krem = fold_and_scan(krem)
        thr_m = prefix_val | chosen4

        def fixup(i, _, cnt2=cnt2, thr_m=thr_m):
            valid = cnt2 > i
            pos = plsc.load_gather(cidx2_v, [lane_cap + i], mask=valid)
            pos = jnp.where(valid, pos, jnp.int32(0))
            xi = plsc.load_gather(buf_v, [pos], mask=valid)
            m = xi ^ ((xi >> 31) & jnp.int32(0x7FFFFFFF))
            rej = jnp.logical_and(m < thr_m, valid)
            plsc.store_scatter(buf_v, [pos], zeros16, mask=rej)
            return _

        lax.fori_loop(0, maxcnt2, fixup, jnp.int32(0))
        pltpu.sync_copy(buf_v, out_hbm.at[row])


def kernel(inputs):
    rows, features = inputs.shape
    k = max(int(round(RATIO * features)), 1)
    xi = lax.bitcast_convert_type(inputs, jnp.int32)
    mesh = plsc.VectorSubcoreMesh(core_axis_name="c", subcore_axis_name="s")
    out_i = pl.kernel(
        functools.partial(_sc_kwta, k=k),
        mesh=mesh,
        compiler_params=pltpu.CompilerParams(needs_layout_passes=False),
        out_type=jax.ShapeDtypeStruct((rows, features), jnp.int32),
        scratch_types=[
            pltpu.VMEM((FEAT,), jnp.int32),
            pltpu.VMEM((FEAT,), jnp.int32),
            pltpu.VMEM((FEAT,), jnp.int32),
            pltpu.VMEM((4096,), jnp.int32),
            pltpu.VMEM((256,), jnp.int32),
            pltpu.SemaphoreType.DMA,
        ],
    )(xi)
    return lax.bitcast_convert_type(out_i, jnp.float32)
